# Initial kernel scaffold; baseline (speedup 1.0000x reference)
import jax
import jax.numpy as jnp
from jax.experimental import pallas as pl


def kernel(x, router_w, router_b, w1, b1, w2, b2, ln_w, ln_b):
    raise NotImplementedError("write your pallas kernel here")


# speedup vs baseline: 5.0552x; 5.0552x over previous
import functools

import jax
import jax.numpy as jnp
from jax import lax
from jax.experimental import pallas as pl
from jax.experimental.pallas import tpu as pltpu
from jax.experimental.pallas import tpu_sc as plsc

D_IN = 1024
D_HID = 4096
D_OUT = 1024
N_EXP = 8
TOP_K = 2
T_TOK = 4096
CAP = max(int(T_TOK * 1.5 / N_EXP), TOP_K)
N_SLOT = T_TOK * TOP_K
N_ROWS = N_EXP * CAP

SC_CORES = 2
SC_SUBCORES = 16
SC_WORKERS = SC_CORES * SC_SUBCORES


def _router_body(x_ref, rw_ref, rb_ref, pf_ref, ei_ref):
    logits = lax.dot_general(
        rw_ref[...], x_ref[...], (((1,), (1,)), ((), ())),
        preferred_element_type=jnp.float32)
    logits = logits + rb_ref[...].reshape(N_EXP, 1)
    m = jnp.max(logits, axis=0, keepdims=True)
    ex = jnp.exp(logits - m)
    z = jnp.sum(ex, axis=0, keepdims=True)
    probs = ex / z
    eidx = lax.broadcasted_iota(jnp.int32, (N_EXP, T_TOK), 0)
    p1 = jnp.max(probs, axis=0, keepdims=True)
    i1 = jnp.min(jnp.where(probs == p1, eidx, N_EXP), axis=0, keepdims=True)
    probs2 = jnp.where(eidx == i1, -jnp.inf, probs)
    p2 = jnp.max(probs2, axis=0, keepdims=True)
    i2 = jnp.min(jnp.where(probs2 == p2, eidx, N_EXP), axis=0, keepdims=True)
    s = p1 + p2
    pf_ref[...] = jnp.concatenate([p1 / s, p2 / s], axis=0)
    ei_ref[...] = jnp.concatenate([i1, i2], axis=0)


def _router(x_flat, router_w, router_b):
    return pl.pallas_call(
        _router_body,
        out_shape=(jax.ShapeDtypeStruct((TOP_K, T_TOK), jnp.float32),
                   jax.ShapeDtypeStruct((TOP_K, T_TOK), jnp.int32)),
    )(x_flat, router_w, router_b)


_R128 = 128
_R64 = N_SLOT // _R128
_RALL = N_EXP * _R64


def _select_body(pf_ref, ei_ref, buf_ref, w_ref):
    pfg = pf_ref[...]
    eig = ei_ref[...]
    bits = lax.bitcast_convert_type(pfg, jnp.int32)
    row = lax.broadcasted_iota(jnp.int32, (_R64, _R128), 0)
    lane = lax.broadcasted_iota(jnp.int32, (_R64, _R128), 1)
    g = row * _R128 + lane
    t = g & (T_TOK - 1)
    k = g >> 12
    inv = (N_SLOT - 1) - (2 * t + k)

    e_iota = lax.broadcasted_iota(jnp.int32, (N_EXP, _R64, _R128), 0)
    mask = eig[None] == e_iota
    pri_b = jnp.where(mask, bits[None], -1)

    def _count(arr, thr):
        c = jnp.sum((arr > thr).astype(jnp.int32), axis=2, keepdims=True)
        return jnp.sum(c, axis=1, keepdims=True)

    def _ph1(_, lh):
        lo, hi = lh
        mid = (lo + hi) >> 1
        small = _count(pri_b, mid) < CAP
        return (jnp.where(small, lo, mid + 1), jnp.where(small, mid, hi))

    lo0 = jnp.zeros((N_EXP, 1, 1), jnp.int32)
    hi0 = jnp.full((N_EXP, 1, 1), 1 << 30, jnp.int32)
    _, hstar = lax.fori_loop(0, 31, _ph1, (lo0, hi0))
    n_gt = _count(pri_b, hstar)
    eq = mask & (bits[None] == hstar)
    pri_l = jnp.where(eq, inv[None], -1)
    target = CAP - n_gt

    def _ph2(_, lh):
        lo, hi = lh
        mid = (lo + hi) >> 1
        small = _count(pri_l, mid) < target
        return (jnp.where(small, lo, mid + 1), jnp.where(small, mid, hi))

    hi1 = jnp.full((N_EXP, 1, 1), N_SLOT, jnp.int32)
    _, lstar = lax.fori_loop(0, 14, _ph2, (lo0, hi1))
    keep_e = mask & ((bits[None] > hstar) | (eq & (inv[None] >= lstar)))

    kf2 = keep_e.astype(jnp.float32).reshape(_RALL, _R128)
    l_i = lax.broadcasted_iota(jnp.int32, (_R128, _R128), 0)
    m_i = lax.broadcasted_iota(jnp.int32, (_R128, _R128), 1)
    tri = (l_i < m_i).astype(jnp.float32)
    within = jnp.dot(kf2, tri, preferred_element_type=jnp.float32)
    ones = jnp.ones((_R128, _R128), jnp.float32)
    rowsum = jnp.dot(kf2, ones, preferred_element_type=jnp.float32)
    r_i = lax.broadcasted_iota(jnp.int32, (_RALL, _RALL), 0)
    s_i = lax.broadcasted_iota(jnp.int32, (_RALL, _RALL), 1)
    blk = (((r_i >> 6) == (s_i >> 6)) & (s_i < r_i)).astype(jnp.float32)
    offs = jnp.dot(blk, rowsum, preferred_element_type=jnp.float32)
    pos = (within + offs).astype(jnp.int32).reshape(N_EXP, _R64, _R128)

    posg = jnp.sum(jnp.where(keep_e, pos, 0), axis=0)
    keep = jnp.any(keep_e, axis=0)
    trash = N_ROWS + (g & 2047)
    buf_ref[...] = jnp.where(keep, eig * CAP + posg, trash)
    w_ref[...] = jnp.where(keep, pfg, 0.0)


def _select(pfg, eig):
    return pl.pallas_call(
        _select_body,
        out_shape=(jax.ShapeDtypeStruct((_R64, _R128), jnp.int32),
                   jax.ShapeDtypeStruct((_R64, _R128), jnp.float32)),
    )(pfg, eig)


_S1_PW = N_SLOT // SC_WORKERS


@functools.lru_cache(maxsize=None)
def _sc_mesh():
    return plsc.VectorSubcoreMesh(core_axis_name="c", subcore_axis_name="s",
                                  num_cores=SC_CORES,
                                  num_subcores=SC_SUBCORES)


@functools.lru_cache(maxsize=None)
def _s1_scatter():
    def body(buf_hbm, out_hbm, idx_v, val_v, sem):
        wid = lax.axis_index("s") * SC_CORES + lax.axis_index("c")
        base = wid * _S1_PW
        for j in range(2):
            pltpu.sync_copy(buf_hbm.at[pl.ds(base + j * 128, 128)],
                            idx_v.at[j])
            for i in range(8):
                off = base + j * 128 + i * 16
                val_v[j, pl.ds(i * 16, 16)] = (
                    (off + lax.iota(jnp.int32, 16)) & (T_TOK - 1))
        for j in range(2):
            pltpu.async_copy(val_v.at[j], out_hbm.at[idx_v.at[j]], sem).wait()

    return pl.kernel(
        body,
        out_type=jax.ShapeDtypeStruct((N_SLOT,), jnp.int32),
        mesh=_sc_mesh(),
        scratch_types=[pltpu.VMEM((2, 128), jnp.int32),
                       pltpu.VMEM((2, 128), jnp.int32),
                       pltpu.SemaphoreType.DMA],
    )


@functools.lru_cache(maxsize=None)
def _make_gather(n_out, n_tab, d):
    per_w = n_out // SC_WORKERS
    n_ch = per_w // 64

    def body(idx_hbm, tab_hbm, out_hbm, idx_v, rows_v, sem):
        wid = lax.axis_index("s") * SC_CORES + lax.axis_index("c")
        base = wid * per_w
        pltpu.sync_copy(idx_hbm.at[pl.ds(base, per_w)], idx_v)
        for i in range(per_w // 16):
            v = idx_v[pl.ds(i * 16, 16)]
            idx_v[pl.ds(i * 16, 16)] = jnp.minimum(
                jnp.maximum(v, 0), n_tab - 1)
        for ch in range(n_ch):
            pltpu.async_copy(
                tab_hbm.at[idx_v.at[pl.ds(ch * 64, 64)]], rows_v, sem).wait()
            pltpu.sync_copy(rows_v, out_hbm.at[pl.ds(base + ch * 64, 64)])

    return pl.kernel(
        body,
        out_type=jax.ShapeDtypeStruct((n_out, d), jnp.float32),
        mesh=_sc_mesh(),
        scratch_types=[pltpu.VMEM((per_w,), jnp.int32),
                       pltpu.VMEM((64, d), jnp.float32),
                       pltpu.SemaphoreType.DMA],
    )


_HB = 512
_NH = D_HID // _HB


def _ffn_body(xe_ref, w1_ref, b1_ref, w2_ref, b2_ref, out_ref, acc_ref):
    h_id = pl.program_id(1)

    @pl.when(h_id == 0)
    def _():
        acc_ref[...] = jnp.zeros_like(acc_ref)

    h = jnp.dot(xe_ref[...], w1_ref[0], preferred_element_type=jnp.float32)
    h = h + b1_ref[0]
    h = h * 0.5 * (1.0 + lax.erf(h * (2.0 ** -0.5)))
    acc_ref[...] += jnp.dot(h, w2_ref[0], preferred_element_type=jnp.float32)

    @pl.when(h_id == _NH - 1)
    def _():
        out_ref[...] = acc_ref[...] + b2_ref[0]


def _ffn(xe, w1, b1, w2, b2):
    return pl.pallas_call(
        _ffn_body,
        grid=(N_EXP, _NH),
        in_specs=[
            pl.BlockSpec((CAP, D_IN), lambda e, h: (e, 0)),
            pl.BlockSpec((1, D_IN, _HB), lambda e, h: (e, 0, h)),
            pl.BlockSpec((1, 1, _HB), lambda e, h: (e, 0, h)),
            pl.BlockSpec((1, _HB, D_OUT), lambda e, h: (e, h, 0)),
            pl.BlockSpec((1, 1, D_OUT), lambda e, h: (e, 0, 0)),
        ],
        out_specs=pl.BlockSpec((CAP, D_OUT), lambda e, h: (e, 0)),
        out_shape=jax.ShapeDtypeStruct((N_ROWS, D_OUT), jnp.float32),
        scratch_shapes=[pltpu.VMEM((CAP, D_OUT), jnp.float32)],
    )(xe, w1, b1.reshape(N_EXP, 1, D_HID), w2, b2.reshape(N_EXP, 1, D_OUT))


_CB = 256


def _combine_body(y0_ref, y1_ref, w0_ref, w1_ref, lnw_ref, lnb_ref, out_ref):
    v = w0_ref[...] * y0_ref[...] + w1_ref[...] * y1_ref[...]
    mu = jnp.mean(v, axis=-1, keepdims=True)
    var = jnp.mean((v - mu) ** 2, axis=-1, keepdims=True)
    out_ref[...] = (v - mu) / jnp.sqrt(var + 1e-5) * lnw_ref[...] + lnb_ref[...]


def _combine(y0, y1, w0, w1c, ln_w, ln_b):
    return pl.pallas_call(
        _combine_body,
        grid=(T_TOK // _CB,),
        in_specs=[
            pl.BlockSpec((_CB, D_OUT), lambda r: (r, 0)),
            pl.BlockSpec((_CB, D_OUT), lambda r: (r, 0)),
            pl.BlockSpec((_CB, 1), lambda r: (r, 0)),
            pl.BlockSpec((_CB, 1), lambda r: (r, 0)),
            pl.BlockSpec((1, D_OUT), lambda r: (0, 0)),
            pl.BlockSpec((1, D_OUT), lambda r: (0, 0)),
        ],
        out_specs=pl.BlockSpec((_CB, D_OUT), lambda r: (r, 0)),
        out_shape=jax.ShapeDtypeStruct((T_TOK, D_OUT), jnp.float32),
    )(y0, y1, w0, w1c, ln_w, ln_b)


def kernel(x, router_w, router_b, w1, b1, w2, b2, ln_w, ln_b):
    bx, sx, d = x.shape
    x_flat = x.reshape(bx * sx, d)
    pf2, ei2 = _router(x_flat, router_w, router_b)
    bufg, wg = _select(pf2.reshape(_R64, _R128), ei2.reshape(_R64, _R128))
    buf_flat = bufg.reshape(N_SLOT)
    src_token = _s1_scatter()(buf_flat)
    xe = _make_gather(N_ROWS, T_TOK, D_IN)(src_token, x_flat)
    oe = _ffn(xe, w1, b1, w2, b2)
    y = _make_gather(N_SLOT, N_ROWS, D_OUT)(buf_flat, oe)
    wt = wg.reshape(TOP_K, T_TOK)
    out = _combine(y[:T_TOK], y[T_TOK:], wt[0][:, None], wt[1][:, None],
                   ln_w.reshape(1, D_OUT), ln_b.reshape(1, D_OUT))
    return out.reshape(bx, sx, D_OUT)
